# baseline (speedup 1.0000x reference)
import functools
import math

import jax
import jax.numpy as jnp
from jax import lax
from jax.experimental import pallas as pl
from jax.experimental.pallas import tpu as pltpu
from jax.experimental.pallas import tpu_sc as plsc

B = 2
L = 2048
D_MODEL = 2048
N_HEADS = 16
D_K = D_MODEL // N_HEADS
KMAX = 10
MIN_FACTOR = 3
MAX_FACTOR = 10
R = N_HEADS * KMAX

ROW_TILE = 512
N_ROW_TILES = (B * L) // ROW_TILE


def _proj_q_kernel(x_ref, w_ref, b_ref, q_ref, s_ref, xbf_ref):
    xt = x_ref[...]
    xbf_ref[...] = xt.astype(jnp.bfloat16)
    q = lax.dot_general(xt, w_ref[...], (((1,), (1,)), ((), ())),
                        preferred_element_type=jnp.float32)
    q = q + b_ref[...]
    q_ref[...] = q
    cols = []
    for h in range(N_HEADS):
        qh = q[:, h * D_K:(h + 1) * D_K]
        l2 = jnp.sqrt(jnp.sum(qh * qh, axis=1, keepdims=True))
        mx = jnp.max(qh, axis=1, keepdims=True)
        t = qh - mx
        e = jnp.exp(t)
        z = jnp.sum(e, axis=1, keepdims=True)
        ent = jnp.log(z) - jnp.sum(e * t, axis=1, keepdims=True) / z
        mean = jnp.mean(qh, axis=1, keepdims=True)
        var = jnp.sum((qh - mean) ** 2, axis=1, keepdims=True) / (D_K - 1)
        cols.append(0.5 * l2 + 0.3 * ent + 0.2 * var)
    s_ref[...] = jnp.concatenate(cols, axis=1)


SC_LANES = 16
SC_CHUNKS = L // SC_LANES
NEG_F32 = -3.0e38


def _sc_topk_kernel(st_ref, idx_ref, u_ref, sv_ref, iv_ref, uv_ref):
    c = lax.axis_index("c")
    s = lax.axis_index("s")
    w = s * 2 + c
    lane = lax.iota(jnp.int32, 16)

    pltpu.sync_copy(st_ref.at[w], sv_ref)

    @pl.when(w % N_HEADS == 0)
    def _stats():
        def sum_body(j, acc):
            v = sv_ref[pl.ds(pl.multiple_of(j * SC_LANES, SC_LANES),
                             SC_LANES)]
            return acc + v

        zeros = jnp.zeros((16,), jnp.float32)
        psum = lax.fori_loop(0, SC_CHUNKS, sum_body, zeros, unroll=8)
        tsum = psum[0]
        for l in range(1, SC_LANES):
            tsum = tsum + psum[l]
        mean_v = jnp.full((16,), tsum) / float(L)

        def var_body(j, acc):
            v = sv_ref[pl.ds(pl.multiple_of(j * SC_LANES, SC_LANES),
                             SC_LANES)]
            d = v - mean_v
            return acc + d * d

        pvar = lax.fori_loop(0, SC_CHUNKS, var_body, zeros, unroll=8)
        tvar = pvar[0]
        for l in range(1, SC_LANES):
            tvar = tvar + pvar[l]
        tvar_v = jnp.full((16,), tvar) / float(L - 1)
        thr = (lane.astype(jnp.float32) + 0.5) * (
            (mean_v + 1e-6) * (1.0 / MAX_FACTOR))
        cntv = jnp.where(tvar_v >= thr * thr,
                         jnp.full((16,), 1.0, jnp.float32),
                         jnp.zeros((16,), jnp.float32))
        cnt = cntv[0]
        for l in range(1, SC_LANES):
            cnt = cnt + cntv[l]
        u = jnp.clip(cnt, float(MIN_FACTOR), float(MAX_FACTOR))
        uv_ref[...] = jnp.full((16,), u)
        pltpu.sync_copy(uv_ref, u_ref.at[w // N_HEADS])

    idx_acc = jnp.zeros((16,), jnp.int32)
    neg = jnp.full((16,), NEG_F32, jnp.float32)
    for i in range(KMAX):
        def max_body(j, carry):
            m, mi = carry
            v = sv_ref[pl.ds(pl.multiple_of(j * SC_LANES, SC_LANES),
                             SC_LANES)]
            upd = v > m
            mi = jnp.where(upd, jnp.full((16,), 0, jnp.int32) + j, mi)
            m = jnp.where(upd, v, m)
            return m, mi

        m, mi = lax.fori_loop(0, SC_CHUNKS, max_body,
                              (neg, jnp.zeros((16,), jnp.int32)), unroll=8)
        gmax = m[0]
        for l in range(1, SC_LANES):
            gmax = jnp.maximum(gmax, m[l])
        cand = jnp.where(m == jnp.full((16,), gmax),
                         mi * SC_LANES + lane, L)
        g = cand[0]
        for l in range(1, SC_LANES):
            g = jnp.minimum(g, cand[l])
        idx_acc = jnp.where(lane == i, jnp.full((16,), 0, jnp.int32) + g,
                            idx_acc)
        off = pl.multiple_of((g // SC_LANES) * SC_LANES, SC_LANES)
        v = sv_ref[pl.ds(off, SC_LANES)]
        sv_ref[pl.ds(off, SC_LANES)] = jnp.where(lane == g % SC_LANES,
                                                 NEG_F32, v)
    iv_ref[...] = idx_acc
    pltpu.sync_copy(iv_ref, idx_ref.at[w])


def _attn_scores_kernel(ids_ref, q_ref, xbf_ref, wk_ref, bk_ref, y_ref,
                        qbd_ref, qsem):
    b = pl.program_id(0)
    qbd_ref[...] = jnp.zeros((R, D_MODEL), jnp.float32)
    copies = []
    for h in range(N_HEADS):
        for i in range(KMAX):
            r = KMAX * h + i
            c = pltpu.make_async_copy(
                q_ref.at[b, pl.ds(ids_ref[N_HEADS * b + h, i], 1),
                         pl.ds(h * D_K, D_K)],
                qbd_ref.at[pl.ds(r, 1), pl.ds(h * D_K, D_K)],
                qsem)
            c.start()
            copies.append(c)
    for c in copies:
        c.wait()
    qbd = qbd_ref[...]
    z = lax.dot_general(qbd, wk_ref[...], (((1,), (0,)), ((), ())),
                        preferred_element_type=jnp.float32)
    sbias = jnp.sum(qbd * bk_ref[...], axis=1, keepdims=True)
    xs = xbf_ref[0]
    s = lax.dot_general(z.astype(jnp.bfloat16), xs, (((1,), (1,)), ((), ())),
                        preferred_element_type=jnp.float32)
    s = (s + sbias) * (1.0 / math.sqrt(D_K))
    s = s - jnp.max(s, axis=1, keepdims=True)
    e = jnp.exp(s)
    a = e / jnp.sum(e, axis=1, keepdims=True)
    y_ref[0] = lax.dot_general(a.astype(jnp.bfloat16), xs,
                               (((1,), (0,)), ((), ())),
                               preferred_element_type=jnp.float32)


L_TILE = 512
N_L_TILES = L // L_TILE


def _proj_scatter_kernel(y_ref, wv_ref, bv_ref, wo_ref, idxpack_ref,
                         u_ref, bo_ref, out_ref, p_scr):
    b = pl.program_id(0)
    lt = pl.program_id(1)

    @pl.when(lt == 0)
    def _compute_p():
        outf = lax.dot_general(y_ref[0], wv_ref[...],
                               (((1,), (1,)), ((), ())),
                               preferred_element_type=jnp.float32)
        outf = outf + bv_ref[...]
        row_h = lax.broadcasted_iota(jnp.int32, (R, D_MODEL), 0) // KMAX
        col_h = lax.broadcasted_iota(jnp.int32, (R, D_MODEL), 1) // D_K
        outbd = jnp.where(row_h == col_h, outf, 0.0)
        p_scr[...] = lax.dot_general(outbd, wo_ref[...],
                                     (((1,), (1,)), ((), ())),
                                     preferred_element_type=jnp.float32)

    lane_i = (lax.broadcasted_iota(jnp.int32, (1, R), 1)
              % KMAX).astype(jnp.float32)
    validrow = (lane_i < u_ref[b, 0]).astype(jnp.float32)
    iol = lax.broadcasted_iota(jnp.int32, (L_TILE, R), 0) + lt * L_TILE
    gt = jnp.where(iol == idxpack_ref[0], validrow, 0.0)
    out_ref[0] = lax.dot_general(gt, p_scr[...], (((1,), (0,)), ((), ())),
                                 preferred_element_type=jnp.float32
                                 ) + bo_ref[...]


def kernel(x, Wq, bq, Wk, bk, Wv, bv, Wo, bo):
    xf = x.reshape(B * L, D_MODEL)
    bq2 = bq.reshape(1, D_MODEL)
    bk2 = bk.reshape(1, D_MODEL)
    bv2 = bv.reshape(1, D_MODEL)
    bo2 = bo.reshape(1, D_MODEL)

    row_spec = pl.BlockSpec((ROW_TILE, D_MODEL), lambda i: (i, 0))
    w_spec = pl.BlockSpec((D_MODEL, D_MODEL), lambda i: (0, 0))
    b_spec = pl.BlockSpec((1, D_MODEL), lambda i: (0, 0))

    q, scores, xbf = pl.pallas_call(
        _proj_q_kernel,
        grid=(N_ROW_TILES,),
        in_specs=[row_spec, w_spec, b_spec],
        out_specs=[row_spec,
                   pl.BlockSpec((ROW_TILE, N_HEADS), lambda i: (i, 0)),
                   row_spec],
        out_shape=[jax.ShapeDtypeStruct((B * L, D_MODEL), jnp.float32),
                   jax.ShapeDtypeStruct((B * L, N_HEADS), jnp.float32),
                   jax.ShapeDtypeStruct((B * L, D_MODEL), jnp.bfloat16)],
    )(xf, Wq, bq2)

    scores_t = (scores.reshape(B, L, N_HEADS).transpose(0, 2, 1)
                .reshape(B * N_HEADS, L))

    sc_mesh = plsc.VectorSubcoreMesh(core_axis_name="c", subcore_axis_name="s")
    idx_sc, u_sc = pl.kernel(
        _sc_topk_kernel,
        mesh=sc_mesh,
        out_type=[jax.ShapeDtypeStruct((B * N_HEADS, 16), jnp.int32),
                  jax.ShapeDtypeStruct((B, 16), jnp.float32)],
        scratch_types=[pltpu.VMEM((L,), jnp.float32),
                       pltpu.VMEM((16,), jnp.int32),
                       pltpu.VMEM((16,), jnp.float32)],
    )(scores_t)

    idxpack = idx_sc.reshape(B, N_HEADS, 16)[:, :, :KMAX].reshape(B, 1, R)

    q3 = q.reshape(B, L, D_MODEL)

    xbf3 = xbf.reshape(B, L, D_MODEL)

    y = pl.pallas_call(
        _attn_scores_kernel,
        grid=(B,),
        in_specs=[pl.BlockSpec(memory_space=pltpu.MemorySpace.SMEM),
                  pl.BlockSpec(memory_space=pltpu.MemorySpace.HBM),
                  pl.BlockSpec((1, L, D_MODEL), lambda b: (b, 0, 0)),
                  pl.BlockSpec((D_MODEL, D_MODEL), lambda b: (0, 0)),
                  pl.BlockSpec((1, D_MODEL), lambda b: (0, 0))],
        out_specs=pl.BlockSpec((1, R, D_MODEL), lambda b: (b, 0, 0)),
        out_shape=jax.ShapeDtypeStruct((B, R, D_MODEL), jnp.float32),
        scratch_shapes=[pltpu.VMEM((R, D_MODEL), jnp.float32),
                        pltpu.SemaphoreType.DMA],
    )(idx_sc, q3, xbf3, Wk, bk2)

    out = pl.pallas_call(
        _proj_scatter_kernel,
        grid=(B, N_L_TILES),
        in_specs=[pl.BlockSpec((1, R, D_MODEL), lambda b, j: (b, 0, 0)),
                  pl.BlockSpec((D_MODEL, D_MODEL), lambda b, j: (0, 0)),
                  pl.BlockSpec((1, D_MODEL), lambda b, j: (0, 0)),
                  pl.BlockSpec((D_MODEL, D_MODEL), lambda b, j: (0, 0)),
                  pl.BlockSpec((1, 1, R), lambda b, j: (b, 0, 0)),
                  pl.BlockSpec(memory_space=pltpu.MemorySpace.SMEM),
                  pl.BlockSpec((1, D_MODEL), lambda b, j: (0, 0))],
        out_specs=pl.BlockSpec((1, L_TILE, D_MODEL), lambda b, j: (b, j, 0)),
        out_shape=jax.ShapeDtypeStruct((B, L, D_MODEL), jnp.float32),
        scratch_shapes=[pltpu.VMEM((R, D_MODEL), jnp.float32)],
    )(y, Wv, bv2, Wo, idxpack, u_sc, bo2)

    return out


# speedup vs baseline: 1.0011x; 1.0011x over previous
import math

import jax
import jax.numpy as jnp
from jax import lax
from jax.experimental import pallas as pl
from jax.experimental.pallas import tpu as pltpu
from jax.experimental.pallas import tpu_sc as plsc

B = 2
L = 2048
D_MODEL = 2048
N_HEADS = 16
D_K = D_MODEL // N_HEADS
KMAX = 10
MIN_FACTOR = 3
MAX_FACTOR = 10
R = N_HEADS * KMAX

ROW_TILE = 512
N_ROW_TILES = (B * L) // ROW_TILE


def _proj_q_kernel(x_ref, w_ref, b_ref, q_ref, s_ref, xbf_ref):
    xt = x_ref[...]
    xbf_ref[...] = xt.astype(jnp.bfloat16)
    q = lax.dot_general(xt, w_ref[...], (((1,), (1,)), ((), ())),
                        preferred_element_type=jnp.float32)
    q = q + b_ref[...]
    q_ref[...] = q
    cols = []
    for h in range(N_HEADS):
        qh = q[:, h * D_K:(h + 1) * D_K]
        l2 = jnp.sqrt(jnp.sum(qh * qh, axis=1, keepdims=True))
        mx = jnp.max(qh, axis=1, keepdims=True)
        t = qh - mx
        e = jnp.exp(t)
        z = jnp.sum(e, axis=1, keepdims=True)
        ent = jnp.log(z) - jnp.sum(e * t, axis=1, keepdims=True) / z
        mean = jnp.mean(qh, axis=1, keepdims=True)
        var = jnp.sum((qh - mean) ** 2, axis=1, keepdims=True) / (D_K - 1)
        cols.append(0.5 * l2 + 0.3 * ent + 0.2 * var)
    s_ref[...] = jnp.concatenate(cols, axis=1)


SC_LANES = 16
SC_CHUNKS = L // SC_LANES
NEG_F32 = -3.0e38


def _sc_topk_kernel(st_ref, idx_ref, u_ref, sv_ref, iv_ref, uv_ref):
    c = lax.axis_index("c")
    s = lax.axis_index("s")
    w = s * 2 + c
    lane = lax.iota(jnp.int32, 16)

    pltpu.sync_copy(st_ref.at[w], sv_ref)

    @pl.when(w % N_HEADS == 0)
    def _stats():
        def sum_body(j, acc):
            v = sv_ref[pl.ds(pl.multiple_of(j * SC_LANES, SC_LANES),
                             SC_LANES)]
            return acc + v

        zeros = jnp.zeros((16,), jnp.float32)
        psum = lax.fori_loop(0, SC_CHUNKS, sum_body, zeros, unroll=8)
        tsum = psum[0]
        for l in range(1, SC_LANES):
            tsum = tsum + psum[l]
        mean_v = jnp.full((16,), tsum) / float(L)

        def var_body(j, acc):
            v = sv_ref[pl.ds(pl.multiple_of(j * SC_LANES, SC_LANES),
                             SC_LANES)]
            d = v - mean_v
            return acc + d * d

        pvar = lax.fori_loop(0, SC_CHUNKS, var_body, zeros, unroll=8)
        tvar = pvar[0]
        for l in range(1, SC_LANES):
            tvar = tvar + pvar[l]
        tvar_v = jnp.full((16,), tvar) / float(L - 1)
        thr = (lane.astype(jnp.float32) + 0.5) * (
            (mean_v + 1e-6) * (1.0 / MAX_FACTOR))
        cntv = jnp.where(tvar_v >= thr * thr,
                         jnp.full((16,), 1.0, jnp.float32),
                         jnp.zeros((16,), jnp.float32))
        cnt = cntv[0]
        for l in range(1, SC_LANES):
            cnt = cnt + cntv[l]
        u = jnp.clip(cnt, float(MIN_FACTOR), float(MAX_FACTOR))
        uv_ref[...] = jnp.full((16,), u)
        pltpu.sync_copy(uv_ref, u_ref.at[w // N_HEADS])

    idx_acc = jnp.zeros((16,), jnp.int32)
    neg = jnp.full((16,), NEG_F32, jnp.float32)
    for i in range(KMAX):
        def max_body(j, carry):
            m, mi = carry
            v = sv_ref[pl.ds(pl.multiple_of(j * SC_LANES, SC_LANES),
                             SC_LANES)]
            upd = v > m
            mi = jnp.where(upd, jnp.full((16,), 0, jnp.int32) + j, mi)
            m = jnp.where(upd, v, m)
            return m, mi

        m, mi = lax.fori_loop(0, SC_CHUNKS, max_body,
                              (neg, jnp.zeros((16,), jnp.int32)), unroll=8)
        gmax = m[0]
        for l in range(1, SC_LANES):
            gmax = jnp.maximum(gmax, m[l])
        cand = jnp.where(m == jnp.full((16,), gmax),
                         mi * SC_LANES + lane, L)
        g = cand[0]
        for l in range(1, SC_LANES):
            g = jnp.minimum(g, cand[l])
        idx_acc = jnp.where(lane == i, jnp.full((16,), 0, jnp.int32) + g,
                            idx_acc)
        off = pl.multiple_of((g // SC_LANES) * SC_LANES, SC_LANES)
        v = sv_ref[pl.ds(off, SC_LANES)]
        sv_ref[pl.ds(off, SC_LANES)] = jnp.where(lane == g % SC_LANES,
                                                 NEG_F32, v)
    iv_ref[...] = idx_acc
    pltpu.sync_copy(iv_ref, idx_ref.at[w])


def _attn_scores_kernel(ids_ref, q_ref, xbf_ref, wk_ref, bk_ref, y_ref,
                        qbd_ref, qsem):
    b = pl.program_id(0)
    qbd_ref[...] = jnp.zeros((R, D_MODEL), jnp.float32)
    copies = []
    for h in range(N_HEADS):
        for i in range(KMAX):
            r = KMAX * h + i
            c = pltpu.make_async_copy(
                q_ref.at[b, pl.ds(ids_ref[N_HEADS * b + h, i], 1),
                         pl.ds(h * D_K, D_K)],
                qbd_ref.at[pl.ds(r, 1), pl.ds(h * D_K, D_K)],
                qsem)
            c.start()
            copies.append(c)
    for c in copies:
        c.wait()
    qbd = qbd_ref[...]
    z = lax.dot_general(qbd, wk_ref[...], (((1,), (0,)), ((), ())),
                        preferred_element_type=jnp.float32)
    sbias = jnp.sum(qbd * bk_ref[...], axis=1, keepdims=True)
    xs = xbf_ref[0]
    s = lax.dot_general(z.astype(jnp.bfloat16), xs, (((1,), (1,)), ((), ())),
                        preferred_element_type=jnp.float32)
    s = (s + sbias) * (1.0 / math.sqrt(D_K))
    s = s - jnp.max(s, axis=1, keepdims=True)
    e = jnp.exp(s)
    a = e / jnp.sum(e, axis=1, keepdims=True)
    y_ref[0] = lax.dot_general(a.astype(jnp.bfloat16), xs,
                               (((1,), (0,)), ((), ())),
                               preferred_element_type=jnp.float32)


L_TILE = 512
N_L_TILES = L // L_TILE


def _proj_scatter_kernel(y_ref, wv_ref, bv_ref, wo_ref, idxpack_ref,
                         u_ref, bo_ref, out_ref, p_scr):
    b = pl.program_id(0)
    lt = pl.program_id(1)

    @pl.when(lt == 0)
    def _compute_p():
        outf = lax.dot_general(y_ref[0], wv_ref[...],
                               (((1,), (1,)), ((), ())),
                               preferred_element_type=jnp.float32)
        outf = outf + bv_ref[...]
        row_h = lax.broadcasted_iota(jnp.int32, (R, D_MODEL), 0) // KMAX
        col_h = lax.broadcasted_iota(jnp.int32, (R, D_MODEL), 1) // D_K
        outbd = jnp.where(row_h == col_h, outf, 0.0)
        p_scr[...] = lax.dot_general(outbd, wo_ref[...],
                                     (((1,), (1,)), ((), ())),
                                     preferred_element_type=jnp.float32)

    lane_i = (lax.broadcasted_iota(jnp.int32, (1, R), 1)
              % KMAX).astype(jnp.float32)
    validrow = (lane_i < u_ref[b, 0]).astype(jnp.float32)
    iol = lax.broadcasted_iota(jnp.int32, (L_TILE, R), 0) + lt * L_TILE
    gt = jnp.where(iol == idxpack_ref[0], validrow, 0.0)
    out_ref[0] = lax.dot_general(gt, p_scr[...], (((1,), (0,)), ((), ())),
                                 preferred_element_type=jnp.float32
                                 ) + bo_ref[...]


def kernel(x, Wq, bq, Wk, bk, Wv, bv, Wo, bo):
    xf = x.reshape(B * L, D_MODEL)
    bq2 = bq.reshape(1, D_MODEL)
    bk2 = bk.reshape(1, D_MODEL)
    bv2 = bv.reshape(1, D_MODEL)
    bo2 = bo.reshape(1, D_MODEL)

    row_spec = pl.BlockSpec((ROW_TILE, D_MODEL), lambda i: (i, 0))
    w_spec = pl.BlockSpec((D_MODEL, D_MODEL), lambda i: (0, 0))
    b_spec = pl.BlockSpec((1, D_MODEL), lambda i: (0, 0))

    q, scores, xbf = pl.pallas_call(
        _proj_q_kernel,
        grid=(N_ROW_TILES,),
        in_specs=[row_spec, w_spec, b_spec],
        out_specs=[row_spec,
                   pl.BlockSpec((ROW_TILE, N_HEADS), lambda i: (i, 0)),
                   row_spec],
        out_shape=[jax.ShapeDtypeStruct((B * L, D_MODEL), jnp.float32),
                   jax.ShapeDtypeStruct((B * L, N_HEADS), jnp.float32),
                   jax.ShapeDtypeStruct((B * L, D_MODEL), jnp.bfloat16)],
    )(xf, Wq, bq2)

    scores_t = (scores.reshape(B, L, N_HEADS).transpose(0, 2, 1)
                .reshape(B * N_HEADS, L))

    sc_mesh = plsc.VectorSubcoreMesh(core_axis_name="c", subcore_axis_name="s")
    idx_sc, u_sc = pl.kernel(
        _sc_topk_kernel,
        mesh=sc_mesh,
        out_type=[jax.ShapeDtypeStruct((B * N_HEADS, 16), jnp.int32),
                  jax.ShapeDtypeStruct((B, 16), jnp.float32)],
        scratch_types=[pltpu.VMEM((L,), jnp.float32),
                       pltpu.VMEM((16,), jnp.int32),
                       pltpu.VMEM((16,), jnp.float32)],
    )(scores_t)

    idxpack = idx_sc.reshape(B, N_HEADS, 16)[:, :, :KMAX].reshape(B, 1, R)

    q3 = q.reshape(B, L, D_MODEL)

    xbf3 = xbf.reshape(B, L, D_MODEL)

    y = pl.pallas_call(
        _attn_scores_kernel,
        grid=(B,),
        in_specs=[pl.BlockSpec(memory_space=pltpu.MemorySpace.SMEM),
                  pl.BlockSpec(memory_space=pltpu.MemorySpace.HBM),
                  pl.BlockSpec((1, L, D_MODEL), lambda b: (b, 0, 0)),
                  pl.BlockSpec((D_MODEL, D_MODEL), lambda b: (0, 0)),
                  pl.BlockSpec((1, D_MODEL), lambda b: (0, 0))],
        out_specs=pl.BlockSpec((1, R, D_MODEL), lambda b: (b, 0, 0)),
        out_shape=jax.ShapeDtypeStruct((B, R, D_MODEL), jnp.float32),
        scratch_shapes=[pltpu.VMEM((R, D_MODEL), jnp.float32),
                        pltpu.SemaphoreType.DMA],
    )(idx_sc, q3, xbf3, Wk, bk2)

    out = pl.pallas_call(
        _proj_scatter_kernel,
        grid=(B, N_L_TILES),
        in_specs=[pl.BlockSpec((1, R, D_MODEL), lambda b, j: (b, 0, 0)),
                  pl.BlockSpec((D_MODEL, D_MODEL), lambda b, j: (0, 0)),
                  pl.BlockSpec((1, D_MODEL), lambda b, j: (0, 0)),
                  pl.BlockSpec((D_MODEL, D_MODEL), lambda b, j: (0, 0)),
                  pl.BlockSpec((1, 1, R), lambda b, j: (b, 0, 0)),
                  pl.BlockSpec(memory_space=pltpu.MemorySpace.SMEM),
                  pl.BlockSpec((1, D_MODEL), lambda b, j: (0, 0))],
        out_specs=pl.BlockSpec((1, L_TILE, D_MODEL), lambda b, j: (b, j, 0)),
        out_shape=jax.ShapeDtypeStruct((B, L, D_MODEL), jnp.float32),
        scratch_shapes=[pltpu.VMEM((R, D_MODEL), jnp.float32)],
    )(y, Wv, bv2, Wo, idxpack, u_sc, bo2)

    return out
